# initial kernel scaffold (unmeasured)
import jax
import jax.numpy as jnp
from jax import lax
from jax.experimental import pallas as pl
from jax.experimental.pallas import tpu as pltpu


def kernel(
    x,
):
    def body(*refs):
        pass

    out_shape = jax.ShapeDtypeStruct(..., jnp.float32)
    return pl.pallas_call(body, out_shape=out_shape)(...)


# baseline (device time: 21583 ns/iter reference)
import jax
import jax.numpy as jnp
from jax import lax
from jax.experimental import pallas as pl
from jax.experimental.pallas import tpu as pltpu

N_DEV = 32


def kernel(x):
    m, n = x.shape

    def body(x_ref, out_ref, comm_ref, send_sems, recv_sems):
        my = lax.axis_index("i")

        total = jnp.sum(x_ref[:, :], axis=0)
        comm_ref[0, :] = total

        rdmas = []
        for d in range(1, N_DEV):
            target = lax.rem(my + d, N_DEV)
            rdma = pltpu.make_async_remote_copy(
                src_ref=comm_ref.at[pl.ds(0, 1)],
                dst_ref=comm_ref.at[pl.ds(d, 1)],
                send_sem=send_sems.at[d],
                recv_sem=recv_sems.at[d],
                device_id=(target,),
                device_id_type=pl.DeviceIdType.MESH,
            )
            rdma.start()
            rdmas.append(rdma)

        row = lax.broadcasted_iota(jnp.int32, (m, m), 0)
        col = lax.broadcasted_iota(jnp.int32, (m, m), 1)
        tri = (row >= col).astype(jnp.float32)
        local_cs = jnp.dot(tri, x_ref[:, :], preferred_element_type=jnp.float32)

        for rdma in rdmas:
            rdma.wait_recv()

        slot = lax.broadcasted_iota(jnp.int32, (N_DEV, 1), 0)
        mask = ((slot >= 1) & (slot <= my)).astype(jnp.float32)
        offset = jnp.sum(comm_ref[:, :] * mask, axis=0, keepdims=True)

        out_ref[:, :] = local_cs + offset

        for rdma in rdmas:
            rdma.wait_send()

    return pl.pallas_call(
        body,
        out_shape=jax.ShapeDtypeStruct((m, n), jnp.float32),
        in_specs=[pl.BlockSpec(memory_space=pltpu.VMEM)],
        out_specs=pl.BlockSpec(memory_space=pltpu.VMEM),
        scratch_shapes=[
            pltpu.VMEM((N_DEV, n), jnp.float32),
            pltpu.SemaphoreType.DMA((N_DEV,)),
            pltpu.SemaphoreType.DMA((N_DEV,)),
        ],
    )(x)


# device time: 13894 ns/iter; 1.5534x vs baseline; 1.5534x over previous
import jax
import jax.numpy as jnp
from jax import lax
from jax.experimental import pallas as pl
from jax.experimental.pallas import tpu as pltpu

N_DEV = 32


def kernel(x):
    m, n = x.shape

    def body(x_ref, out_ref, comm_ref, send_sems, recv_sems):
        my = lax.axis_index("i")

        barrier_sem = pltpu.get_barrier_semaphore()
        for d in range(1, N_DEV):
            pl.semaphore_signal(
                barrier_sem,
                inc=1,
                device_id=(lax.rem(my + d, N_DEV),),
                device_id_type=pl.DeviceIdType.MESH,
            )
        pl.semaphore_wait(barrier_sem, N_DEV - 1)

        total = jnp.sum(x_ref[:, :], axis=0)
        comm_ref[0, :] = total

        rdmas = []
        for d in range(1, N_DEV):
            target = lax.rem(my + d, N_DEV)
            rdma = pltpu.make_async_remote_copy(
                src_ref=comm_ref.at[pl.ds(0, 1)],
                dst_ref=comm_ref.at[pl.ds(d, 1)],
                send_sem=send_sems.at[d],
                recv_sem=recv_sems.at[d],
                device_id=(target,),
                device_id_type=pl.DeviceIdType.MESH,
            )
            rdma.start()
            rdmas.append(rdma)

        row = lax.broadcasted_iota(jnp.int32, (m, m), 0)
        col = lax.broadcasted_iota(jnp.int32, (m, m), 1)
        tri = (row >= col).astype(jnp.float32)
        local_cs = jnp.dot(tri, x_ref[:, :], preferred_element_type=jnp.float32)

        for rdma in rdmas:
            rdma.wait_recv()

        slot = lax.broadcasted_iota(jnp.int32, (N_DEV, 1), 0)
        mask = ((slot >= 1) & (slot <= my)).astype(jnp.float32)
        offset = jnp.sum(comm_ref[:, :] * mask, axis=0, keepdims=True)

        out_ref[:, :] = local_cs + offset

        for rdma in rdmas:
            rdma.wait_send()

    return pl.pallas_call(
        body,
        out_shape=jax.ShapeDtypeStruct((m, n), jnp.float32),
        in_specs=[pl.BlockSpec(memory_space=pltpu.VMEM)],
        out_specs=pl.BlockSpec(memory_space=pltpu.VMEM),
        scratch_shapes=[
            pltpu.VMEM((N_DEV, n), jnp.float32),
            pltpu.SemaphoreType.DMA((N_DEV,)),
            pltpu.SemaphoreType.DMA((N_DEV,)),
        ],
        compiler_params=pltpu.CompilerParams(collective_id=0),
    )(x)


# device time: 12720 ns/iter; 1.6968x vs baseline; 1.0923x over previous
import jax
import jax.numpy as jnp
from jax import lax
from jax.experimental import pallas as pl
from jax.experimental.pallas import tpu as pltpu

N_DEV = 32


def kernel(x):
    m, n = x.shape

    def body(x_ref, out_ref, comm_ref, send_sems, recv_sems):
        my = lax.axis_index("i")

        barrier_sem = pltpu.get_barrier_semaphore()
        for d in range(1, N_DEV):
            @pl.when(d <= my)
            def _(d=d):
                pl.semaphore_signal(
                    barrier_sem,
                    inc=1,
                    device_id=(my - d,),
                    device_id_type=pl.DeviceIdType.MESH,
                )
        for d in range(1, N_DEV):
            @pl.when(my + d <= N_DEV - 1)
            def _():
                pl.semaphore_wait(barrier_sem, 1)

        comm_ref[0, :] = jnp.sum(x_ref[:, :], axis=0)

        rdmas = []
        for d in range(1, N_DEV):
            sends = my + d <= N_DEV - 1
            target = jnp.minimum(my + d, N_DEV - 1)
            rdma = pltpu.make_async_remote_copy(
                src_ref=comm_ref.at[pl.ds(0, 1)],
                dst_ref=comm_ref.at[pl.ds(d, 1)],
                send_sem=send_sems.at[d],
                recv_sem=recv_sems.at[d],
                device_id=(target,),
                device_id_type=pl.DeviceIdType.MESH,
            )
            @pl.when(sends)
            def _(rdma=rdma):
                rdma.start()
            rdmas.append((rdma, sends))

        row = lax.broadcasted_iota(jnp.int32, (m, m), 0)
        col = lax.broadcasted_iota(jnp.int32, (m, m), 1)
        tri = (row >= col).astype(jnp.float32)
        local_cs = jnp.dot(tri, x_ref[:, :], preferred_element_type=jnp.float32)

        for d in range(1, N_DEV):
            @pl.when(d <= my)
            def _(rdma=rdmas[d - 1][0]):
                rdma.wait_recv()

        slot = lax.broadcasted_iota(jnp.int32, (N_DEV, 1), 0)
        mask = ((slot >= 1) & (slot <= my)).astype(jnp.float32)
        offset = jnp.sum(comm_ref[:, :] * mask, axis=0, keepdims=True)

        out_ref[:, :] = local_cs + offset

        for rdma, sends in rdmas:
            @pl.when(sends)
            def _(rdma=rdma):
                rdma.wait_send()

    return pl.pallas_call(
        body,
        out_shape=jax.ShapeDtypeStruct((m, n), jnp.float32),
        in_specs=[pl.BlockSpec(memory_space=pltpu.VMEM)],
        out_specs=pl.BlockSpec(memory_space=pltpu.VMEM),
        scratch_shapes=[
            pltpu.VMEM((N_DEV, n), jnp.float32),
            pltpu.SemaphoreType.DMA((N_DEV,)),
            pltpu.SemaphoreType.DMA((N_DEV,)),
        ],
        compiler_params=pltpu.CompilerParams(collective_id=0),
    )(x)
